# baseline (speedup 1.0000x reference)
import functools

import jax
import jax.numpy as jnp
from jax import lax
from jax.experimental import pallas as pl
from jax.experimental.pallas import tpu as pltpu
from jax.experimental.pallas import tpu_sc as plsc

_NB = 4


def _off_slices(base, S, CW):
    starts, mapping, cur = [], [], None
    for t in range(S):
        p = base + t
        if cur is None or p - cur >= 16:
            cur = min((p // 8) * 8, CW - 16)
            starts.append(cur)
        mapping.append((len(starts) - 1, p - cur))
    return starts, mapping


def _build(B, S, V, D):
    NC, NS = 2, 16
    NW = NC * NS
    SP = S + (-S) % 4
    CW = 2 * SP
    W = 4 * D
    assert CW <= 128 and D == 32 and B % (2 * NW * _NB) == 0 and V % 4 == 0
    CPW = B // (2 * NW)
    RPW = B // NW
    scale = 1.0 / B

    mesh = plsc.VectorSubcoreMesh(core_axis_name="c", subcore_axis_name="s")

    @functools.partial(
        pl.kernel,
        mesh=mesh,
        out_type=jax.ShapeDtypeStruct((B, D), jnp.float32),
        scratch_types=[
            pltpu.VMEM((CPW, CW), jnp.int32),
            pltpu.VMEM((CPW, CW), jnp.int32),
            [pltpu.VMEM((CW, W), jnp.float32)] * _NB,
            pltpu.VMEM((RPW, D), jnp.float32),
            [pltpu.SemaphoreType.DMA] * _NB,
        ],
    )
    def k(gidx_hbm, goff_hbm, t4_hbm, out_hbm, idx_v, off_v, slabs, out_v, sems):
        wid = lax.axis_index("s") * NC + lax.axis_index("c")
        pltpu.sync_copy(gidx_hbm.at[pl.ds(wid * CPW, CPW)], idx_v)
        pltpu.sync_copy(goff_hbm.at[pl.ds(wid * CPW, CPW)], off_v)

        for b in range(_NB):
            pltpu.async_copy(t4_hbm.at[idx_v.at[b]], slabs[b], sems[b])

        def body(i, _):
            j0 = i * _NB
            for b in range(_NB):
                j = j0 + b
                sv = slabs[b]
                pltpu.make_async_copy(
                    t4_hbm.at[idx_v.at[j]], sv, sems[b]).wait()
                for h in range(2):
                    base = h * SP
                    starts, tmap = _off_slices(base, S, CW)
                    ovs = [off_v[j, pl.ds(s, 16)] for s in starts]
                    ev = [None, None]
                    od = [None, None]
                    for t in range(S):
                        si, lane = tmap[t]
                        off = ovs[si][lane]
                        tgt = ev if t % 2 == 0 else od
                        for d in range(2):
                            v = sv[base + t, pl.ds(off + 16 * d, 16)]
                            tgt[d] = v if tgt[d] is None else tgt[d] + v
                    out_v[2 * j + h, pl.ds(0, 16)] = (ev[0] + od[0]) * scale
                    out_v[2 * j + h, pl.ds(16, 16)] = (ev[1] + od[1]) * scale

                nj = j + _NB

                @pl.when(nj < CPW)
                def _():
                    pltpu.async_copy(t4_hbm.at[idx_v.at[nj]], sv, sems[b])

            return 0

        lax.fori_loop(0, CPW // _NB, body, 0)
        pltpu.sync_copy(out_v, out_hbm.at[pl.ds(wid * RPW, RPW)])

    return k


def kernel(document, table):
    B, S = document.shape
    V, D = table.shape
    SP = S + (-S) % 4
    doc_p = jnp.pad(document, ((0, 0), (0, SP - S)))
    gidx = (doc_p >> 2).reshape(B // 2, 2 * SP)
    goff = ((doc_p & 3) * D).reshape(B // 2, 2 * SP)
    t4 = table.reshape(V // 4, 4 * D)
    return _build(B, S, V, D)(gidx, goff, t4)
